# baseline (speedup 1.0000x reference)
import functools

import jax
import jax.numpy as jnp
from jax import lax
from jax.experimental import pallas as pl
from jax.experimental.pallas import tpu as pltpu
from jax.experimental.pallas import tpu_sc as plsc

_CHUNK = 128
_L = 16


def _build_sc_kernel(B, Dx, Dy, NC, NS):
    NW = NC * NS
    b_per_w = B // NW
    n_chunks = b_per_w // _CHUNK
    D = Dx + Dy
    mesh = plsc.VectorSubcoreMesh(core_axis_name="c", subcore_axis_name="s")

    @functools.partial(
        pl.kernel,
        mesh=mesh,
        compiler_params=pltpu.CompilerParams(needs_layout_passes=False),
        out_type=jax.ShapeDtypeStruct((B, D), jnp.float32),
        scratch_types=[
            pltpu.VMEM((b_per_w,), jnp.int32),
            pltpu.VMEM((b_per_w,), jnp.int32),
            pltpu.VMEM((n_chunks, _CHUNK), jnp.int32),
            pltpu.VMEM((n_chunks, _CHUNK), jnp.int32),
            pltpu.VMEM((n_chunks, _CHUNK), jnp.int32),
            pltpu.VMEM((n_chunks, _CHUNK), jnp.int32),
            pltpu.VMEM((_CHUNK, 128), jnp.float32),
            pltpu.VMEM((_CHUNK, 128), jnp.float32),
            pltpu.VMEM((b_per_w, D), jnp.float32),
            pltpu.SemaphoreType.DMA,
        ],
    )
    def k(xi_hbm, yi_hbm, wx_hbm, wy_hbm, out_hbm,
          xi_v, yi_v, lix_v, liy_v, sox_v, soy_v, lx_v, ly_v, cat_v, sem):
        wid = lax.axis_index("s") * NC + lax.axis_index("c")
        base = wid * b_per_w
        pltpu.sync_copy(xi_hbm.at[pl.ds(base, b_per_w)], xi_v)
        pltpu.sync_copy(yi_hbm.at[pl.ds(base, b_per_w)], yi_v)

        lanes = lax.iota(jnp.int32, _L)

        def split(kk, carry):
            j = kk // (_CHUNK // _L)
            col = lax.rem(kk, _CHUNK // _L) * _L
            for raw_v, li_v, so_v in ((xi_v, lix_v, sox_v), (yi_v, liy_v, soy_v)):
                raw = raw_v[pl.ds(kk * _L, _L)]
                li_v[j, pl.ds(col, _L)] = lax.shift_right_logical(raw, 2)
                so_v[j, pl.ds(col, _L)] = lax.shift_left(
                    lax.bitwise_and(raw, 3), 5)
            return carry
        lax.fori_loop(0, b_per_w // _L, split, 0)

        def do_chunk(j, carry):
            cx = pltpu.async_copy(wx_hbm.at[lix_v.at[j]], lx_v, sem)
            cy = pltpu.async_copy(wy_hbm.at[liy_v.at[j]], ly_v, sem)
            cx.wait()
            cy.wait()

            def group(g, carry2):
                rloc = g * _L + lanes
                rglob = j * _CHUNK + rloc
                socx = sox_v[j, pl.ds(g * _L, _L)]
                socy = soy_v[j, pl.ds(g * _L, _L)]
                for c in range(Dx):
                    v = plsc.load_gather(lx_v, [rloc, socx + c])
                    plsc.store_scatter(
                        cat_v, [rglob, jnp.full((_L,), c, jnp.int32)], v)
                for c in range(Dy):
                    v = plsc.load_gather(ly_v, [rloc, socy + c])
                    plsc.store_scatter(
                        cat_v, [rglob, jnp.full((_L,), Dx + c, jnp.int32)], v)
                return carry2
            lax.fori_loop(0, _CHUNK // _L, group, 0)
            return carry
        lax.fori_loop(0, n_chunks, do_chunk, 0)

        pltpu.sync_copy(cat_v, out_hbm.at[pl.ds(base, b_per_w)])

    return k


def kernel(x, Wx, Wy):
    B = x.shape[0]
    Vx, Dx = Wx.shape
    Vy, Dy = Wy.shape
    info = plsc.get_sparse_core_info()
    k = _build_sc_kernel(B, Dx, Dy, info.num_cores, info.num_subcores)
    x32 = x.astype(jnp.int32)
    wx4 = Wx.reshape(Vx // 4, 4 * Dx)
    wy4 = Wy.reshape(Vy // 4, 4 * Dy)
    return k(x32[:, 0], x32[:, 1], wx4, wy4)


# speedup vs baseline: 1.3500x; 1.3500x over previous
import functools

import jax
import jax.numpy as jnp
from jax import lax
from jax.experimental import pallas as pl
from jax.experimental.pallas import tpu as pltpu
from jax.experimental.pallas import tpu_sc as plsc

_L = 16


def _build_sc_kernel(B, Dx, Dy, NC, NS):
    NW = NC * NS
    b_per_w = B // NW
    D = Dx + Dy
    mesh = plsc.VectorSubcoreMesh(core_axis_name="c", subcore_axis_name="s")

    @functools.partial(
        pl.kernel,
        mesh=mesh,
        compiler_params=pltpu.CompilerParams(needs_layout_passes=False),
        out_type=jax.ShapeDtypeStruct((B, D), jnp.float32),
        scratch_types=[
            pltpu.VMEM((b_per_w,), jnp.int32),
            pltpu.VMEM((b_per_w,), jnp.int32),
            pltpu.VMEM((_L, 8, Dx), jnp.float32),
            pltpu.VMEM((_L, 8, Dy), jnp.float32),
            pltpu.VMEM((b_per_w, D), jnp.float32),
            pltpu.SemaphoreType.DMA,
        ],
    )
    def k(xi_hbm, yi_hbm, wx_hbm, wy_hbm, out_hbm, xi_v, yi_v, bx_v, by_v, cat_v, sem):
        wid = lax.axis_index("s") * NC + lax.axis_index("c")
        base = wid * b_per_w
        pltpu.sync_copy(xi_hbm.at[pl.ds(base, b_per_w)], xi_v)
        pltpu.sync_copy(yi_hbm.at[pl.ds(base, b_per_w)], yi_v)

        def blk(g, carry):
            idxv = xi_v[pl.ds(g * _L, _L)]
            idyv = yi_v[pl.ds(g * _L, _L)]
            for kk in range(_L):
                r0 = pl.multiple_of(
                    lax.shift_left(lax.shift_right_logical(idxv[kk], 3), 3), 8)
                pltpu.async_copy(wx_hbm.at[pl.ds(r0, 8)], bx_v.at[kk], sem)
            for kk in range(_L):
                r0 = pl.multiple_of(
                    lax.shift_left(lax.shift_right_logical(idyv[kk], 3), 3), 8)
                pltpu.async_copy(wy_hbm.at[pl.ds(r0, 8)], by_v.at[kk], sem)
            for kk in range(_L):
                pltpu.make_async_copy(
                    wx_hbm.at[pl.ds(0, 8)], bx_v.at[kk], sem).wait()
                pltpu.make_async_copy(
                    wy_hbm.at[pl.ds(0, 8)], by_v.at[kk], sem).wait()
            sub = lax.bitwise_and(idxv, 7)
            suby = lax.bitwise_and(idyv, 7)
            lanes = lax.iota(jnp.int32, _L)
            for c in range(Dx):
                v = plsc.load_gather(
                    bx_v, [lanes, sub, jnp.full((_L,), c, jnp.int32)])
                plsc.store_scatter(
                    cat_v, [g * _L + lanes, jnp.full((_L,), c, jnp.int32)], v)
            for c in range(Dy):
                v = plsc.load_gather(
                    by_v, [lanes, suby, jnp.full((_L,), c, jnp.int32)])
                plsc.store_scatter(
                    cat_v, [g * _L + lanes, jnp.full((_L,), Dx + c, jnp.int32)], v)
            return carry
        lax.fori_loop(0, b_per_w // _L, blk, 0)

        pltpu.sync_copy(cat_v, out_hbm.at[pl.ds(base, b_per_w)])

    return k


def kernel(x, Wx, Wy):
    B = x.shape[0]
    Dx = Wx.shape[1]
    Dy = Wy.shape[1]
    info = plsc.get_sparse_core_info()
    k = _build_sc_kernel(B, Dx, Dy, info.num_cores, info.num_subcores)
    x32 = x.astype(jnp.int32)
    return k(x32[:, 0], x32[:, 1], Wx, Wy)


# speedup vs baseline: 1.4118x; 1.0458x over previous
import functools

import jax
import jax.numpy as jnp
from jax import lax
from jax.experimental import pallas as pl
from jax.experimental.pallas import tpu as pltpu
from jax.experimental.pallas import tpu_sc as plsc

_L = 16


def _build_sc_kernel(B, Dx, Dy, NC, NS):
    NW = NC * NS
    b_per_w = B // NW
    n_groups = b_per_w // _L
    D = Dx + Dy
    mesh = plsc.VectorSubcoreMesh(core_axis_name="c", subcore_axis_name="s")

    @functools.partial(
        pl.kernel,
        mesh=mesh,
        compiler_params=pltpu.CompilerParams(needs_layout_passes=False),
        out_type=jax.ShapeDtypeStruct((B, D), jnp.float32),
        scratch_types=[
            pltpu.VMEM((b_per_w,), jnp.int32),
            pltpu.VMEM((b_per_w,), jnp.int32),
            pltpu.VMEM((2, _L, 8, Dx), jnp.float32),
            pltpu.VMEM((2, _L, 8, Dy), jnp.float32),
            pltpu.VMEM((8 * _L, D), jnp.float32),
            pltpu.SemaphoreType.DMA,
            pltpu.SemaphoreType.DMA,
        ],
    )
    def k(xi_hbm, yi_hbm, wx_hbm, wy_hbm, out_hbm,
          xi_v, yi_v, bx_v, by_v, cat_v, sem0, sem1):
        wx3 = wx_hbm.reshape(wx_hbm.shape[0] // 8, 8, Dx)
        wy3 = wy_hbm.reshape(wy_hbm.shape[0] // 8, 8, Dy)
        wid = lax.axis_index("s") * NC + lax.axis_index("c")
        base = wid * b_per_w
        pltpu.sync_copy(xi_hbm.at[pl.ds(base, b_per_w)], xi_v)
        pltpu.sync_copy(yi_hbm.at[pl.ds(base, b_per_w)], yi_v)

        lanes = lax.iota(jnp.int32, _L)

        def fire(g, par, sem):
            idxv = xi_v[pl.ds(g * _L, _L)]
            idyv = yi_v[pl.ds(g * _L, _L)]
            for kk in range(_L):
                pltpu.async_copy(
                    wx3.at[lax.shift_right_logical(idxv[kk], 3)],
                    bx_v.at[par, kk], sem)
            for kk in range(_L):
                pltpu.async_copy(
                    wy3.at[lax.shift_right_logical(idyv[kk], 3)],
                    by_v.at[par, kk], sem)

        def wait(par, sem):
            pltpu.make_async_copy(wx3.at[pl.ds(0, _L)], bx_v.at[par], sem).wait()
            pltpu.make_async_copy(wy3.at[pl.ds(0, _L)], by_v.at[par], sem).wait()

        def extract(g, par, row0):
            subx = lax.bitwise_and(xi_v[pl.ds(g * _L, _L)], 7)
            suby = lax.bitwise_and(yi_v[pl.ds(g * _L, _L)], 7)
            rcat = row0 + lanes
            pv = jnp.full((_L,), par, jnp.int32)
            for c in range(Dx):
                v = plsc.load_gather(
                    bx_v, [pv, lanes, subx, jnp.full((_L,), c, jnp.int32)])
                plsc.store_scatter(
                    cat_v, [rcat, jnp.full((_L,), c, jnp.int32)], v)
            for c in range(Dy):
                v = plsc.load_gather(
                    by_v, [pv, lanes, suby, jnp.full((_L,), c, jnp.int32)])
                plsc.store_scatter(
                    cat_v, [rcat, jnp.full((_L,), Dx + c, jnp.int32)], v)

        def window(w, carry):
            g0 = 8 * w
            fire(g0, 0, sem0)
            fire(g0 + 1, 1, sem1)
            for u in range(4):
                wait(0, sem0)
                extract(g0 + 2 * u, 0, 2 * u * _L)
                if u < 3:
                    fire(g0 + 2 * u + 2, 0, sem0)
                wait(1, sem1)
                extract(g0 + 2 * u + 1, 1, (2 * u + 1) * _L)
                if u < 3:
                    fire(g0 + 2 * u + 3, 1, sem1)
            pltpu.sync_copy(
                cat_v, out_hbm.at[pl.ds(base + w * 8 * _L, 8 * _L)])
            return carry
        lax.fori_loop(0, n_groups // 8, window, 0)

    return k


def kernel(x, Wx, Wy):
    B = x.shape[0]
    Dx = Wx.shape[1]
    Dy = Wy.shape[1]
    info = plsc.get_sparse_core_info()
    k = _build_sc_kernel(B, Dx, Dy, info.num_cores, info.num_subcores)
    x32 = x.astype(jnp.int32)
    return k(x32[:, 0], x32[:, 1], Wx, Wy)


# speedup vs baseline: 1.5305x; 1.0841x over previous
import functools

import jax
import jax.numpy as jnp
from jax import lax
from jax.experimental import pallas as pl
from jax.experimental.pallas import tpu as pltpu
from jax.experimental.pallas import tpu_sc as plsc

_L = 16


def _build_sc_kernel(B, Dx, Dy, NC, NS):
    NW = NC * NS
    b_per_w = B // NW
    n_groups = b_per_w // _L
    D = Dx + Dy
    mesh = plsc.VectorSubcoreMesh(core_axis_name="c", subcore_axis_name="s")

    @functools.partial(
        pl.kernel,
        mesh=mesh,
        compiler_params=pltpu.CompilerParams(needs_layout_passes=False),
        out_type=jax.ShapeDtypeStruct((B, D), jnp.float32),
        scratch_types=[
            pltpu.VMEM((b_per_w,), jnp.int32),
            pltpu.VMEM((b_per_w,), jnp.int32),
            pltpu.VMEM((2, _L, Dx), jnp.float32),
            pltpu.VMEM((2, _L, Dy), jnp.float32),
            pltpu.VMEM((8 * _L, D), jnp.float32),
            pltpu.SemaphoreType.DMA,
            pltpu.SemaphoreType.DMA,
        ],
    )
    def k(xi_hbm, yi_hbm, wx_hbm, wy_hbm, out_hbm,
          xi_v, yi_v, bx_v, by_v, cat_v, sem0, sem1):
        wx3 = wx_hbm.reshape(wx_hbm.shape[0] // 8, 8, Dx)
        wy3 = wy_hbm.reshape(wy_hbm.shape[0] // 8, 8, Dy)
        wid = lax.axis_index("s") * NC + lax.axis_index("c")
        base = wid * b_per_w
        pltpu.sync_copy(xi_hbm.at[pl.ds(base, b_per_w)], xi_v)
        pltpu.sync_copy(yi_hbm.at[pl.ds(base, b_per_w)], yi_v)

        lanes = lax.iota(jnp.int32, _L)

        def fire(g, par, sem):
            idxv = xi_v[pl.ds(g * _L, _L)]
            idyv = yi_v[pl.ds(g * _L, _L)]
            for kk in range(_L):
                i = idxv[kk]
                pltpu.async_copy(
                    wx3.at[lax.shift_right_logical(i, 3), lax.bitwise_and(i, 7)],
                    bx_v.at[par, kk], sem)
            for kk in range(_L):
                i = idyv[kk]
                pltpu.async_copy(
                    wy3.at[lax.shift_right_logical(i, 3), lax.bitwise_and(i, 7)],
                    by_v.at[par, kk], sem)

        def wait(par, sem):
            pltpu.make_async_copy(
                wx3.at[pl.ds(0, 2), pl.ds(0, 8)].reshape(_L, Dx),
                bx_v.at[par], sem).wait()
            pltpu.make_async_copy(
                wy3.at[pl.ds(0, 2), pl.ds(0, 8)].reshape(_L, Dy),
                by_v.at[par], sem).wait()

        def extract(g, par, row0):
            for kk in range(_L):
                for c0 in range(0, Dx, _L):
                    cat_v[row0 + kk, pl.ds(c0, _L)] = bx_v[par, kk, pl.ds(c0, _L)]
                for c0 in range(0, Dy, _L):
                    cat_v[row0 + kk, pl.ds(Dx + c0, _L)] = by_v[par, kk, pl.ds(c0, _L)]

        def window(w, carry):
            g0 = 8 * w
            fire(g0, 0, sem0)
            fire(g0 + 1, 1, sem1)
            for u in range(4):
                wait(0, sem0)
                extract(g0 + 2 * u, 0, 2 * u * _L)
                if u < 3:
                    fire(g0 + 2 * u + 2, 0, sem0)
                wait(1, sem1)
                extract(g0 + 2 * u + 1, 1, (2 * u + 1) * _L)
                if u < 3:
                    fire(g0 + 2 * u + 3, 1, sem1)
            pltpu.sync_copy(
                cat_v, out_hbm.at[pl.ds(base + w * 8 * _L, 8 * _L)])
            return carry
        lax.fori_loop(0, n_groups // 8, window, 0)

    return k


def kernel(x, Wx, Wy):
    B = x.shape[0]
    Dx = Wx.shape[1]
    Dy = Wy.shape[1]
    info = plsc.get_sparse_core_info()
    k = _build_sc_kernel(B, Dx, Dy, info.num_cores, info.num_subcores)
    x32 = x.astype(jnp.int32)
    return k(x32[:, 0], x32[:, 1], Wx, Wy)


# speedup vs baseline: 1.5338x; 1.0021x over previous
import functools

import jax
import jax.numpy as jnp
from jax import lax
from jax.experimental import pallas as pl
from jax.experimental.pallas import tpu as pltpu
from jax.experimental.pallas import tpu_sc as plsc

_L = 16


def _build_sc_kernel(B, Dx, Dy, NC, NS):
    NW = NC * NS
    b_per_w = B // NW
    n_groups = b_per_w // _L
    D = Dx + Dy
    mesh = plsc.VectorSubcoreMesh(core_axis_name="c", subcore_axis_name="s")

    @functools.partial(
        pl.kernel,
        mesh=mesh,
        compiler_params=pltpu.CompilerParams(
            needs_layout_passes=False, use_tc_tiling_on_sc=True),
        out_type=jax.ShapeDtypeStruct((B, D), jnp.float32),
        scratch_types=[
            pltpu.VMEM((b_per_w,), jnp.int32),
            pltpu.VMEM((b_per_w,), jnp.int32),
            pltpu.VMEM((2, _L, Dx), jnp.float32),
            pltpu.VMEM((2, _L, Dy), jnp.float32),
            pltpu.VMEM((8 * _L, D), jnp.float32),
            pltpu.SemaphoreType.DMA,
            pltpu.SemaphoreType.DMA,
        ],
    )
    def k(xi_hbm, yi_hbm, wx_hbm, wy_hbm, out_hbm,
          xi_v, yi_v, bx_v, by_v, cat_v, sem0, sem1):
        wx3 = wx_hbm.reshape(wx_hbm.shape[0] // 8, 8, Dx)
        wy3 = wy_hbm.reshape(wy_hbm.shape[0] // 8, 8, Dy)
        wid = lax.axis_index("s") * NC + lax.axis_index("c")
        base = wid * b_per_w
        pltpu.sync_copy(xi_hbm.at[pl.ds(base, b_per_w)], xi_v)
        pltpu.sync_copy(yi_hbm.at[pl.ds(base, b_per_w)], yi_v)

        lanes = lax.iota(jnp.int32, _L)

        def fire(g, par, sem):
            idxv = xi_v[pl.ds(g * _L, _L)]
            idyv = yi_v[pl.ds(g * _L, _L)]
            for kk in range(_L):
                i = idxv[kk]
                pltpu.async_copy(
                    wx3.at[lax.shift_right_logical(i, 3), lax.bitwise_and(i, 7)],
                    bx_v.at[par, kk], sem)
            for kk in range(_L):
                i = idyv[kk]
                pltpu.async_copy(
                    wy3.at[lax.shift_right_logical(i, 3), lax.bitwise_and(i, 7)],
                    by_v.at[par, kk], sem)

        def wait(par, sem):
            pltpu.make_async_copy(
                wx3.at[pl.ds(0, 2), pl.ds(0, 8)].reshape(_L, Dx),
                bx_v.at[par], sem).wait()
            pltpu.make_async_copy(
                wy3.at[pl.ds(0, 2), pl.ds(0, 8)].reshape(_L, Dy),
                by_v.at[par], sem).wait()

        def extract(g, par, row0):
            for kk in range(_L):
                for c0 in range(0, Dx, _L):
                    cat_v[row0 + kk, pl.ds(c0, _L)] = bx_v[par, kk, pl.ds(c0, _L)]
                for c0 in range(0, Dy, _L):
                    cat_v[row0 + kk, pl.ds(Dx + c0, _L)] = by_v[par, kk, pl.ds(c0, _L)]

        def window(w, carry):
            g0 = 8 * w
            fire(g0, 0, sem0)
            fire(g0 + 1, 1, sem1)
            for u in range(4):
                wait(0, sem0)
                extract(g0 + 2 * u, 0, 2 * u * _L)
                if u < 3:
                    fire(g0 + 2 * u + 2, 0, sem0)
                wait(1, sem1)
                extract(g0 + 2 * u + 1, 1, (2 * u + 1) * _L)
                if u < 3:
                    fire(g0 + 2 * u + 3, 1, sem1)
            pltpu.sync_copy(
                cat_v, out_hbm.at[pl.ds(base + w * 8 * _L, 8 * _L)])
            return carry
        lax.fori_loop(0, n_groups // 8, window, 0)

    return k


def kernel(x, Wx, Wy):
    B = x.shape[0]
    Dx = Wx.shape[1]
    Dy = Wy.shape[1]
    info = plsc.get_sparse_core_info()
    k = _build_sc_kernel(B, Dx, Dy, info.num_cores, info.num_subcores)
    x32 = x.astype(jnp.int32)
    return k(x32[:, 0], x32[:, 1], Wx, Wy)
